# baseline (speedup 1.0000x reference)
import functools

import jax
import jax.numpy as jnp
from jax import lax
from jax.experimental import pallas as pl
from jax.experimental.pallas import tpu as pltpu
from jax.experimental.pallas import tpu_sc as plsc

VOCAB = 1000000
EMB_DIM = 32
UNITS = 64
BATCH = 16384

BN = 65536
NROW = BN // 128
NBLK = (VOCAB + BN - 1) // BN
FROWS = NBLK * NROW

NC = 2
NS = 16
NW = NC * NS
BPW = BATCH // NW
GV = BPW // 16

BM = 2048


def _mlp_all_body(eT_ref, w1_ref, b1c_ref, w2c_ref, b2_ref, o_ref):
    e = jnp.maximum(eT_ref[...], 0.0).astype(jnp.bfloat16)
    hT = lax.dot_general(w1_ref[...], e, (((0,), (0,)), ((), ())),
                         preferred_element_type=jnp.float32)
    hT = jnp.maximum(hT + b1c_ref[...], 0.0)
    f = jnp.sum(hT * w2c_ref[...], axis=0, keepdims=True)
    f = f + b2_ref[0, 0]
    col = lax.broadcasted_iota(jnp.int32, (1, BN), 1) + pl.program_id(0) * BN
    f = jnp.where(col < VOCAB, f, 0.0)
    for j in range(NROW):
        o_ref[pl.ds(j, 1), :] = f[:, j * 128:(j + 1) * 128]


def _tc_mlp_all(eT, W1, b1c, W2c, b2r):
    return pl.pallas_call(
        _mlp_all_body,
        grid=(NBLK,),
        in_specs=[
            pl.BlockSpec((EMB_DIM, BN), lambda i: (0, i)),
            pl.BlockSpec((EMB_DIM, UNITS), lambda i: (0, 0)),
            pl.BlockSpec((UNITS, 1), lambda i: (0, 0)),
            pl.BlockSpec((UNITS, 1), lambda i: (0, 0)),
            pl.BlockSpec((1, 1), lambda i: (0, 0)),
        ],
        out_specs=pl.BlockSpec((NROW, 128), lambda i: (i, 0)),
        out_shape=jax.ShapeDtypeStruct((FROWS, 128), jnp.float32),
    )(eT, W1.astype(jnp.bfloat16), b1c, W2c, b2r)


@functools.partial(
    pl.kernel,
    out_type=jax.ShapeDtypeStruct((BATCH, 128), jnp.float32),
    mesh=plsc.VectorSubcoreMesh(core_axis_name="c", subcore_axis_name="s"),
    scratch_types=[
        pltpu.VMEM((BPW,), jnp.int32),
        pltpu.VMEM((BPW,), jnp.int32),
        pltpu.VMEM((BPW, 128), jnp.float32),
        pltpu.SemaphoreType.DMA,
    ],
)
def _sc_rowgather(x_hbm, f_hbm, out_hbm, x_v, hi_v, rows_v, sem):
    wid = lax.axis_index("s") * NC + lax.axis_index("c")
    base = wid * BPW
    pltpu.sync_copy(x_hbm.at[wid], x_v)
    for g in range(GV):
        hi_v[pl.ds(g * 16, 16)] = lax.shift_right_logical(
            x_v[pl.ds(g * 16, 16)], 7
        )
    copies = [
        pltpu.async_copy(
            f_hbm.at[hi_v.at[pl.ds(j * 128, 128)]],
            rows_v.at[pl.ds(j * 128, 128)],
            sem,
        )
        for j in range(BPW // 128)
    ]
    for c in copies:
        c.wait()
    pltpu.sync_copy(rows_v, out_hbm.at[pl.ds(base, BPW)])


def _select_body(x3_ref, rows_ref, o_ref):
    subl = lax.broadcasted_iota(jnp.int32, (128, 128), 0)
    ident = (subl == lax.broadcasted_iota(jnp.int32, (128, 128), 1))
    identf = ident.astype(jnp.float32)
    lo_all = lax.bitwise_and(x3_ref[...], 127)
    for r in range(BM // 128):
        lo = lo_all[r:r + 1, :]
        oh = (subl == lo).astype(jnp.float32)
        m = rows_ref[pl.ds(r * 128, 128), :]
        t = lax.dot_general(m, oh, (((1,), (0,)), ((), ())),
                            preferred_element_type=jnp.float32)
        o_ref[pl.ds(r, 1), :] = jnp.sum(t * identf, axis=0, keepdims=True)


def _tc_select(x3, rows):
    return pl.pallas_call(
        _select_body,
        grid=(BATCH // BM,),
        in_specs=[
            pl.BlockSpec((BM // 128, 128), lambda i: (i, 0)),
            pl.BlockSpec((BM, 128), lambda i: (i, 0)),
        ],
        out_specs=pl.BlockSpec((BM // 128, 128), lambda i: (i, 0)),
        out_shape=jax.ShapeDtypeStruct((BATCH // 128, 128), jnp.float32),
    )(x3, rows)


def kernel(x, emb, W1, b1, W2, b2):
    f2d = _tc_mlp_all(
        emb.T,
        W1,
        b1.reshape(UNITS, 1),
        W2,
        b2.reshape(1, 1),
    )
    xi = x.astype(jnp.int32)
    rows = _sc_rowgather(xi.reshape(NW, BPW), f2d)
    out128 = _tc_select(xi.reshape(BATCH // 128, 128), rows)
    return out128.reshape(BATCH, 1)


# speedup vs baseline: 1.0308x; 1.0308x over previous
import functools

import jax
import jax.numpy as jnp
from jax import lax
from jax.experimental import pallas as pl
from jax.experimental.pallas import tpu as pltpu
from jax.experimental.pallas import tpu_sc as plsc

VOCAB = 1000000
EMB_DIM = 32
UNITS = 64
BATCH = 16384

BN = 32768
NROW = BN // 128
NBLK = (VOCAB + BN - 1) // BN
FROWS = NBLK * NROW

NC = 2
NS = 16
NW = NC * NS
BPW = BATCH // NW
GV = BPW // 16

BM = 2048


def _mlp_all_body(eT_ref, w1_ref, b1c_ref, w2c_ref, b2_ref, o_ref):
    e = jnp.maximum(eT_ref[...], 0.0).astype(jnp.bfloat16)
    hT = lax.dot_general(w1_ref[...], e, (((0,), (0,)), ((), ())),
                         preferred_element_type=jnp.float32)
    hT = jnp.maximum(hT + b1c_ref[...], 0.0)
    f = jnp.sum(hT * w2c_ref[...], axis=0, keepdims=True)
    f = f + b2_ref[0, 0]
    col = lax.broadcasted_iota(jnp.int32, (1, BN), 1) + pl.program_id(0) * BN
    f = jnp.where(col < VOCAB, f, 0.0)
    for j in range(NROW):
        o_ref[pl.ds(j, 1), :] = f[:, j * 128:(j + 1) * 128]


def _tc_mlp_all(eT, W1, b1c, W2c, b2r):
    return pl.pallas_call(
        _mlp_all_body,
        grid=(NBLK,),
        in_specs=[
            pl.BlockSpec((EMB_DIM, BN), lambda i: (0, i)),
            pl.BlockSpec((EMB_DIM, UNITS), lambda i: (0, 0)),
            pl.BlockSpec((UNITS, 1), lambda i: (0, 0)),
            pl.BlockSpec((UNITS, 1), lambda i: (0, 0)),
            pl.BlockSpec((1, 1), lambda i: (0, 0)),
        ],
        out_specs=pl.BlockSpec((NROW, 128), lambda i: (i, 0)),
        out_shape=jax.ShapeDtypeStruct((FROWS, 128), jnp.float32),
    )(eT, W1.astype(jnp.bfloat16), b1c, W2c, b2r)


@functools.partial(
    pl.kernel,
    out_type=jax.ShapeDtypeStruct((BATCH, 128), jnp.float32),
    mesh=plsc.VectorSubcoreMesh(core_axis_name="c", subcore_axis_name="s"),
    scratch_types=[
        pltpu.VMEM((BPW,), jnp.int32),
        pltpu.VMEM((BPW,), jnp.int32),
        pltpu.VMEM((BPW, 128), jnp.float32),
        pltpu.SemaphoreType.DMA,
    ],
)
def _sc_rowgather(x_hbm, f_hbm, out_hbm, x_v, hi_v, rows_v, sem):
    wid = lax.axis_index("s") * NC + lax.axis_index("c")
    base = wid * BPW
    pltpu.sync_copy(x_hbm.at[pl.ds(base, BPW)], x_v)
    for g in range(GV):
        hi_v[pl.ds(g * 16, 16)] = lax.shift_right_logical(
            x_v[pl.ds(g * 16, 16)], 7
        )
    copies = [
        pltpu.async_copy(
            f_hbm.at[hi_v.at[pl.ds(j * 128, 128)]],
            rows_v.at[pl.ds(j * 128, 128)],
            sem,
        )
        for j in range(BPW // 128)
    ]
    for c in copies:
        c.wait()
    pltpu.sync_copy(rows_v, out_hbm.at[pl.ds(base, BPW)])


def _select_body(x3_ref, rows_ref, o_ref):
    subl = lax.broadcasted_iota(jnp.int32, (128, 128), 0)
    ident = (subl == lax.broadcasted_iota(jnp.int32, (128, 128), 1))
    identf = ident.astype(jnp.float32)
    lo_all = lax.bitwise_and(x3_ref[...], 127)
    for r in range(BM // 128):
        lo = lo_all[r:r + 1, :]
        oh = (subl == lo).astype(jnp.float32)
        m = rows_ref[pl.ds(r * 128, 128), :]
        t = lax.dot_general(m, oh, (((1,), (0,)), ((), ())),
                            preferred_element_type=jnp.float32)
        o_ref[pl.ds(r, 1), :] = jnp.sum(t * identf, axis=0, keepdims=True)


def _tc_select(x3, rows):
    return pl.pallas_call(
        _select_body,
        grid=(BATCH // BM,),
        in_specs=[
            pl.BlockSpec((BM // 128, 128), lambda i: (i, 0)),
            pl.BlockSpec((BM, 128), lambda i: (i, 0)),
        ],
        out_specs=pl.BlockSpec((BM // 128, 128), lambda i: (i, 0)),
        out_shape=jax.ShapeDtypeStruct((BATCH // 128, 128), jnp.float32),
    )(x3, rows)


def kernel(x, emb, W1, b1, W2, b2):
    f2d = _tc_mlp_all(
        emb.T,
        W1,
        b1.reshape(UNITS, 1),
        W2,
        b2.reshape(1, 1),
    )
    xi = x.astype(jnp.int32)
    rows = _sc_rowgather(xi, f2d)
    out128 = _tc_select(xi.reshape(BATCH // 128, 128), rows)
    return out128.reshape(BATCH, 1)


# speedup vs baseline: 1.0438x; 1.0126x over previous
import functools

import jax
import jax.numpy as jnp
from jax import lax
from jax.experimental import pallas as pl
from jax.experimental.pallas import tpu as pltpu
from jax.experimental.pallas import tpu_sc as plsc

VOCAB = 1000000
EMB_DIM = 32
UNITS = 64
BATCH = 16384

BN = 32768
NROW = BN // 128
NBLK = (VOCAB + BN - 1) // BN
FROWS = NBLK * NROW

NC = 2
NS = 16
NW = NC * NS
BPW = BATCH // NW
GV = BPW // 16

BM = 2048


def _mlp_all_body(eT_ref, w1_ref, b1c_ref, w2c_ref, b2_ref, o_ref):
    e = jnp.maximum(eT_ref[...], 0.0).astype(jnp.bfloat16)
    w1 = w1_ref[...].astype(jnp.bfloat16)
    hT = lax.dot_general(w1, e, (((0,), (0,)), ((), ())),
                         preferred_element_type=jnp.float32)
    hT = jnp.maximum(hT + b1c_ref[...], 0.0)
    f = jnp.sum(hT * w2c_ref[...], axis=0, keepdims=True)
    f = f + b2_ref[0, 0]
    col = lax.broadcasted_iota(jnp.int32, (1, BN), 1) + pl.program_id(0) * BN
    f = jnp.where(col < VOCAB, f, 0.0)
    for j in range(NROW):
        o_ref[pl.ds(j, 1), :] = f[:, j * 128:(j + 1) * 128]


def _tc_mlp_all(eT, W1, b1c, W2c, b2r):
    return pl.pallas_call(
        _mlp_all_body,
        grid=(NBLK,),
        in_specs=[
            pl.BlockSpec((EMB_DIM, BN), lambda i: (0, i)),
            pl.BlockSpec((EMB_DIM, UNITS), lambda i: (0, 0)),
            pl.BlockSpec((UNITS, 1), lambda i: (0, 0)),
            pl.BlockSpec((UNITS, 1), lambda i: (0, 0)),
            pl.BlockSpec((1, 1), lambda i: (0, 0)),
        ],
        out_specs=pl.BlockSpec((NROW, 128), lambda i: (i, 0)),
        out_shape=jax.ShapeDtypeStruct((FROWS, 128), jnp.float32),
    )(eT, W1, b1c, W2c, b2r)


@functools.partial(
    pl.kernel,
    out_type=jax.ShapeDtypeStruct((BATCH, 128), jnp.float32),
    mesh=plsc.VectorSubcoreMesh(core_axis_name="c", subcore_axis_name="s"),
    scratch_types=[
        pltpu.VMEM((BPW,), jnp.int32),
        pltpu.VMEM((BPW,), jnp.int32),
        pltpu.VMEM((BPW, 128), jnp.float32),
        pltpu.SemaphoreType.DMA,
    ],
)
def _sc_rowgather(x_hbm, f_hbm, out_hbm, x_v, hi_v, rows_v, sem):
    wid = lax.axis_index("s") * NC + lax.axis_index("c")
    base = wid * BPW
    pltpu.sync_copy(x_hbm.at[pl.ds(base, BPW)], x_v)
    for g in range(GV):
        hi_v[pl.ds(g * 16, 16)] = lax.shift_right_logical(
            x_v[pl.ds(g * 16, 16)], 7
        )
    copies = [
        pltpu.async_copy(
            f_hbm.at[hi_v.at[pl.ds(j * 128, 128)]],
            rows_v.at[pl.ds(j * 128, 128)],
            sem,
        )
        for j in range(BPW // 128)
    ]
    for c in copies:
        c.wait()
    pltpu.sync_copy(rows_v, out_hbm.at[pl.ds(base, BPW)])


def _select_body(x3_ref, rows_ref, o_ref):
    subl = lax.broadcasted_iota(jnp.int32, (128, 128), 0)
    ident = (subl == lax.broadcasted_iota(jnp.int32, (128, 128), 1))
    identf = ident.astype(jnp.float32)
    lo_all = lax.bitwise_and(x3_ref[...], 127)
    for r in range(BM // 128):
        lo = lo_all[r:r + 1, :]
        oh = (subl == lo).astype(jnp.float32)
        m = rows_ref[pl.ds(r * 128, 128), :]
        t = lax.dot_general(m, oh, (((1,), (0,)), ((), ())),
                            preferred_element_type=jnp.float32)
        o_ref[pl.ds(r, 1), :] = jnp.sum(t * identf, axis=0, keepdims=True)


def _tc_select(x3, rows):
    return pl.pallas_call(
        _select_body,
        grid=(BATCH // BM,),
        in_specs=[
            pl.BlockSpec((BM // 128, 128), lambda i: (i, 0)),
            pl.BlockSpec((BM, 128), lambda i: (i, 0)),
        ],
        out_specs=pl.BlockSpec((BM // 128, 128), lambda i: (i, 0)),
        out_shape=jax.ShapeDtypeStruct((BATCH // 128, 128), jnp.float32),
    )(x3, rows)


def kernel(x, emb, W1, b1, W2, b2):
    f2d = _tc_mlp_all(
        emb.T,
        W1,
        b1.reshape(UNITS, 1),
        W2,
        b2.reshape(1, 1),
    )
    xi = x.astype(jnp.int32)
    rows = _sc_rowgather(xi, f2d)
    out128 = _tc_select(xi.reshape(BATCH // 128, 128), rows)
    return out128.reshape(BATCH, 1)
